# Initial kernel scaffold; baseline (speedup 1.0000x reference)
import jax
import jax.numpy as jnp
from jax.experimental import pallas as pl


def kernel(params, x_founder, x_university, x_company_size, x_industry, x_role_type, src_studied_at, dst_studied_at, src_worked_at, dst_worked_at, src_in, dst_in, src_had, dst_had):
    raise NotImplementedError("write your pallas kernel here")


# speedup vs baseline: 1.6303x; 1.6303x over previous
import functools

import jax
import jax.numpy as jnp
from jax.experimental import pallas as pl
from jax.experimental.pallas import tpu as pltpu

_H = 128
_NF = 50000
_NU = 10000
_N_SMALL = {"worked_at": 10, "in": 150, "had": 50}


def _mm_kernel(x_ref, w_ref, o_ref, acc_ref, *, nk):
    @pl.when(pl.program_id(2) == 0)
    def _init():
        acc_ref[...] = jnp.zeros_like(acc_ref)

    acc_ref[...] += jnp.dot(x_ref[...], w_ref[...],
                            preferred_element_type=jnp.float32)

    @pl.when(pl.program_id(2) == nk - 1)
    def _fin():
        o_ref[...] = acc_ref[...]


def _ceil_to(x, m):
    return -(-x // m) * m


def _mm(x, w, bm, bn, bk):
    m, k = x.shape
    _, n = w.shape
    mp, kp, np_ = _ceil_to(m, bm), _ceil_to(k, bk), _ceil_to(n, bn)
    if mp > m or kp > k:
        x = jnp.pad(x, ((0, mp - m), (0, kp - k)))
    if kp > k or np_ > n:
        w = jnp.pad(w, ((0, kp - k), (0, np_ - n)))
    nk = kp // bk
    out = pl.pallas_call(
        functools.partial(_mm_kernel, nk=nk),
        grid=(mp // bm, np_ // bn, nk),
        in_specs=[
            pl.BlockSpec((bm, bk), lambda i, j, kk: (i, kk)),
            pl.BlockSpec((bk, bn), lambda i, j, kk: (kk, j)),
        ],
        out_specs=pl.BlockSpec((bm, bn), lambda i, j, kk: (i, j)),
        out_shape=jax.ShapeDtypeStruct((mp, np_), jnp.float32),
        scratch_shapes=[pltpu.VMEM((bm, bn), jnp.float32)],
        compiler_params=pltpu.CompilerParams(
            dimension_semantics=("parallel", "parallel", "arbitrary")),
    )(x, w)
    if mp > m or np_ > n:
        out = out[:m, :n]
    return out


def _mm_big(x, w):
    return _mm(x, w, bm=1024, bn=128, bk=_ceil_to(x.shape[1], 128))


def kernel(params, x_founder, x_university, x_company_size, x_industry,
           x_role_type, src_studied_at, dst_studied_at, src_worked_at,
           dst_worked_at, src_in, dst_in, src_had, dst_had):
    xs = {"founder": x_founder, "university": x_university,
          "company_size": x_company_size, "industry": x_industry,
          "role_type": x_role_type}
    rels = {"studied_at": ("university", src_studied_at, dst_studied_at),
            "worked_at": ("company_size", src_worked_at, dst_worked_at),
            "in": ("industry", src_in, dst_in),
            "had": ("role_type", src_had, dst_had)}

    proj = params["proj"]
    h = {}
    for nt, x in xs.items():
        bm = 1024 if x.shape[0] > 1024 else _ceil_to(x.shape[0], 8)
        h[nt] = _mm(x, proj[nt]["W"], bm=bm, bn=128,
                    bk=_ceil_to(x.shape[1], 128)) + proj[nt]["b"]

    M = {}
    Mt = {}
    for r, (nt, src, dst) in rels.items():
        if nt == "university":
            continue
        ns = _N_SMALL[r]
        M[r] = jnp.zeros((_NF, ns), jnp.float32).at[src, dst].add(1.0)
        Mt[r] = jnp.zeros((ns, _NF), jnp.float32).at[dst, src].add(1.0)
    deg_u = jnp.zeros((_NU,), jnp.float32).at[dst_studied_at].add(1.0)
    deg_fu = jnp.zeros((_NF,), jnp.float32).at[src_studied_at].add(1.0)
    ones_f = jnp.ones((_NF, 1), jnp.float32)

    for layer in params["convs"]:
        h_f_aug = jnp.concatenate([h["founder"], ones_f], axis=1)
        new_h = {}
        founder_acc = []

        for r, (nt, src, dst) in rels.items():
            p_fwd = layer["founder__" + r + "__" + nt]
            p_rev = layer[nt + "__rev_" + r + "__founder"]
            if nt == "university":
                sum_u = jax.ops.segment_sum(
                    jnp.take(h["founder"], src, axis=0), dst,
                    num_segments=_NU)
                aggr_d = sum_u / jnp.clip(deg_u, 1.0, None)[:, None]
                rev_sum = jax.ops.segment_sum(
                    jnp.take(h[nt], dst, axis=0), src, num_segments=_NF)
                aggr_f = rev_sum / jnp.clip(deg_fu, 1.0, None)[:, None]
            else:
                ns = _N_SMALL[r]
                s = _mm(Mt[r], h_f_aug, bm=_ceil_to(ns, 8), bn=128, bk=1024)
                aggr_d = s[:, :_H] / jnp.clip(s[:, _H:_H + 1], 1.0, None)
                h_s_aug = jnp.concatenate(
                    [h[nt], jnp.ones((ns, 1), jnp.float32)], axis=1)
                rsum = _mm(M[r], h_s_aug, bm=1024, bn=128,
                           bk=_ceil_to(ns + 1, 128))
                aggr_f = rsum[:, :_H] / jnp.clip(rsum[:, _H:_H + 1], 1.0,
                                                 None)

            bm_d = 1024 if aggr_d.shape[0] > 1024 else _ceil_to(
                aggr_d.shape[0], 8)
            out_d = (_mm(aggr_d, p_fwd["W_l"], bm=bm_d, bn=128, bk=128)
                     + p_fwd["b_l"]
                     + _mm(h[nt], p_fwd["W_r"], bm=bm_d, bn=128, bk=128))
            new_h[nt] = jax.nn.relu(out_d)
            founder_acc.append((aggr_f, p_rev))

        w_r_mean = sum(p["W_r"] for _, p in founder_acc) * 0.25
        b_l_mean = sum(p["b_l"] for _, p in founder_acc) * 0.25
        out_f = _mm_big(h["founder"], w_r_mean) + b_l_mean
        for aggr_f, p in founder_acc:
            out_f = out_f + 0.25 * _mm_big(aggr_f, p["W_l"])
        new_h["founder"] = jax.nn.relu(out_f)
        h = new_h

    founder = h["founder"]
    cls = params["cls"]
    h1 = jax.nn.relu(_mm(founder, cls["W1"], bm=1024, bn=128, bk=128)
                     + cls["b1"])
    logits = _mm(h1, cls["W2"], bm=1024, bn=128, bk=128)[:, 0] + cls["b2"][0]
    return logits, founder
